# baseline (speedup 1.0000x reference)
import jax
import jax.numpy as jnp
from jax.experimental import pallas as pl
from jax.experimental.pallas import tpu as pltpu

B, T, D_IN, D_OUT, R, E = 32, 1, 4096, 4096, 16, 16
TILE_O = 512
NT = D_OUT // TILE_O
NBUF = 4


def _body(x_ref, ids_ref, a_hbm, w_hbm, b_hbm, out_ref,
          w_buf, a_vmem, b_vmem, mid_ref, w_sems, a_sem, b_sem):
    def w_copy(j, slot):
        return pltpu.make_async_copy(
            w_hbm.at[pl.ds(j * TILE_O, TILE_O), :],
            w_buf.at[slot],
            w_sems.at[slot],
        )

    a_copy = pltpu.make_async_copy(a_hbm, a_vmem, a_sem)
    b_copy = pltpu.make_async_copy(b_hbm, b_vmem, b_sem)
    a_copy.start()
    b_copy.start()
    for s in range(NBUF):
        w_copy(s, s).start()

    a_copy.wait()
    xb = x_ref[...].astype(jnp.bfloat16)
    mid_all = jax.lax.dot_general(
        xb, a_vmem[...].astype(jnp.bfloat16), (((1,), (1,)), ((), ())),
        preferred_element_type=jnp.float32,
    )
    col_e = jax.lax.broadcasted_iota(jnp.int32, (B, E * R), 1) // R
    mid_ref[...] = jnp.where(col_e == ids_ref[...], mid_all, 0.0)

    b_copy.wait()
    out_ref[...] = jax.lax.dot_general(
        mid_ref[...].astype(jnp.bfloat16), b_vmem[...].astype(jnp.bfloat16),
        (((1,), (0,)), ((), ())),
        preferred_element_type=jnp.float32,
    )

    for j in range(NT):
        slot = j % NBUF
        w_copy(j, slot).wait()
        h = jax.lax.dot_general(
            xb, w_buf[slot].astype(jnp.bfloat16), (((1,), (1,)), ((), ())),
            preferred_element_type=jnp.float32,
        )
        nxt = j + NBUF
        if nxt < NT:
            w_copy(nxt, slot).start()
        out_ref[:, pl.ds(j * TILE_O, TILE_O)] += h


@jax.jit
def kernel(x, adapter_ids, W_base, A_all, B_all):
    x2 = x.reshape(B, D_IN)
    a2 = A_all.reshape(E * R, D_IN)
    b_r = jnp.swapaxes(B_all, 1, 2).reshape(E * R, D_OUT)
    ids2 = adapter_ids.reshape(B, 1).astype(jnp.int32)
    out = pl.pallas_call(
        _body,
        in_specs=[
            pl.BlockSpec((B, D_IN), lambda: (0, 0)),
            pl.BlockSpec((B, 1), lambda: (0, 0)),
            pl.BlockSpec(memory_space=pltpu.MemorySpace.HBM),
            pl.BlockSpec(memory_space=pltpu.MemorySpace.HBM),
            pl.BlockSpec(memory_space=pltpu.MemorySpace.HBM),
        ],
        out_specs=pl.BlockSpec((B, D_OUT), lambda: (0, 0)),
        out_shape=jax.ShapeDtypeStruct((B, D_OUT), jnp.float32),
        scratch_shapes=[
            pltpu.VMEM((NBUF, TILE_O, D_IN), jnp.float32),
            pltpu.VMEM((E * R, D_IN), jnp.float32),
            pltpu.VMEM((E * R, D_OUT), jnp.float32),
            pltpu.VMEM((B, E * R), jnp.float32),
            pltpu.SemaphoreType.DMA((NBUF,)),
            pltpu.SemaphoreType.DMA,
            pltpu.SemaphoreType.DMA,
        ],
    )(x2, ids2, a2, W_base, b_r)
    return out.reshape(B, T, D_OUT)


# speedup vs baseline: 1.1424x; 1.1424x over previous
import jax
import jax.numpy as jnp
from jax.experimental import pallas as pl
from jax.experimental.pallas import tpu as pltpu

B, T, D_IN, D_OUT, R, E = 32, 1, 4096, 4096, 16, 16
TILE_O = 512
NT = D_OUT // TILE_O
NBUF = 4


def _body(x_ref, ids_ref, a_hbm, w_hbm, b_hbm, out_ref,
          w_buf, a_vmem, b_vmem, mid_ref, w_sems, a_sem, b_sem):
    def w_copy(j, slot):
        return pltpu.make_async_copy(
            w_hbm.at[pl.ds(j * TILE_O, TILE_O), :],
            w_buf.at[slot],
            w_sems.at[slot],
        )

    a_copy = pltpu.make_async_copy(a_hbm, a_vmem, a_sem)
    b_copy = pltpu.make_async_copy(b_hbm, b_vmem, b_sem)
    a_copy.start()
    b_copy.start()
    for s in range(NBUF):
        w_copy(s, s).start()

    a_copy.wait()
    xb = x_ref[...].astype(jnp.bfloat16)
    mid_all = jax.lax.dot_general(
        x_ref[...], a_vmem[...], (((1,), (1,)), ((), ())),
        preferred_element_type=jnp.float32,
    )
    col_e = jax.lax.broadcasted_iota(jnp.int32, (B, E * R), 1) // R
    mid_ref[...] = jnp.where(col_e == ids_ref[...], mid_all, 0.0)

    b_copy.wait()
    out_ref[...] = jax.lax.dot_general(
        mid_ref[...], b_vmem[...], (((1,), (0,)), ((), ())),
        preferred_element_type=jnp.float32,
    )

    for j in range(NT):
        slot = j % NBUF
        w_copy(j, slot).wait()
        h = jax.lax.dot_general(
            xb, w_buf[slot].astype(jnp.bfloat16), (((1,), (1,)), ((), ())),
            preferred_element_type=jnp.float32,
        )
        nxt = j + NBUF
        if nxt < NT:
            w_copy(nxt, slot).start()
        out_ref[:, pl.ds(j * TILE_O, TILE_O)] += h


@jax.jit
def kernel(x, adapter_ids, W_base, A_all, B_all):
    x2 = x.reshape(B, D_IN)
    a2 = A_all.reshape(E * R, D_IN)
    b_r = jnp.swapaxes(B_all, 1, 2).reshape(E * R, D_OUT)
    ids2 = adapter_ids.reshape(B, 1).astype(jnp.int32)
    out = pl.pallas_call(
        _body,
        in_specs=[
            pl.BlockSpec((B, D_IN), lambda: (0, 0)),
            pl.BlockSpec((B, 1), lambda: (0, 0)),
            pl.BlockSpec(memory_space=pltpu.MemorySpace.HBM),
            pl.BlockSpec(memory_space=pltpu.MemorySpace.HBM),
            pl.BlockSpec(memory_space=pltpu.MemorySpace.HBM),
        ],
        out_specs=pl.BlockSpec((B, D_OUT), lambda: (0, 0)),
        out_shape=jax.ShapeDtypeStruct((B, D_OUT), jnp.float32),
        scratch_shapes=[
            pltpu.VMEM((NBUF, TILE_O, D_IN), jnp.float32),
            pltpu.VMEM((E * R, D_IN), jnp.float32),
            pltpu.VMEM((E * R, D_OUT), jnp.float32),
            pltpu.VMEM((B, E * R), jnp.float32),
            pltpu.SemaphoreType.DMA((NBUF,)),
            pltpu.SemaphoreType.DMA,
            pltpu.SemaphoreType.DMA,
        ],
    )(x2, ids2, a2, W_base, b_r)
    return out.reshape(B, T, D_OUT)
